# baseline (speedup 1.0000x reference)
import functools

import jax
import jax.numpy as jnp
from jax.experimental import pallas as pl
from jax.experimental.pallas import tpu as pltpu

_B = 32
_L = 196
_K = 4096
_INV_TEMP = 10.0
_MOM = 0.75
_SINKHORN_ITERS = 3
_RECON_N = 32 * 3 * 224 * 224
_GRID = 2 * _B
_RCHUNK = _RECON_N // _GRID


def _body(alpha_ref, teacher_ref, student_ref, proto_ref, mask_ref,
          recon_ref, label_ref, out_ref, acc_ref, cst_ref, cnt_ref, rec_ref):
    i = pl.program_id(0)

    @pl.when(i == 0)
    def _init():
        cst_ref[0] = 0.0
        cnt_ref[0] = 0.0
        rec_ref[0] = 0.0
        acc_ref[...] = teacher_ref[0]

    @pl.when(jnp.logical_and(i > 0, i < _B))
    def _teacher_acc():
        acc_ref[...] += teacher_ref[0]

    dr = recon_ref[0] - label_ref[0]
    rec_ref[0] += jnp.sum(dr * dr)

    @pl.when(i == _B)
    def _sinkhorn():
        alpha = alpha_ref[0, 0]
        batch_q = acc_ref[...] * (1.0 / _B)
        proto = alpha * proto_ref[0] + (1.0 - alpha) * batch_q
        a = jnp.exp(proto * 20.0)
        for _ in range(_SINKHORN_ITERS):
            a = a / jnp.sum(a, axis=0, keepdims=True)
            a = a / jnp.sum(a, axis=1, keepdims=True)
        acc_ref[...] = a

    @pl.when(i >= _B)
    def _student():
        x = student_ref[0] * _INV_TEMP
        t = acc_ref[...]
        m = jnp.max(x, axis=-1)
        se = jnp.sum(jnp.exp(x - m[:, None]), axis=-1)
        lse = m + jnp.log(se)
        dot = jnp.sum(t * x, axis=-1)
        mask = mask_ref[0, 0]
        cst_ref[0] += jnp.sum((lse - dot) * mask)
        cnt_ref[0] += jnp.sum(mask)

    @pl.when(i == _GRID - 1)
    def _finish():
        val = cst_ref[0] / cnt_ref[0] + rec_ref[0] * (1.0 / _RECON_N)
        out_ref[...] = jnp.full((1, 1), val, dtype=jnp.float32)


@functools.partial(jax.jit, static_argnames=("interpret",))
def _afire_loss(alpha, teacher_q, student_q, prototype, mask_f, recon2,
                label2, interpret=False):
    out = pl.pallas_call(
        _body,
        grid=(_GRID,),
        in_specs=[
            pl.BlockSpec(memory_space=pltpu.SMEM),
            pl.BlockSpec((1, _L, _K), lambda i: (jnp.minimum(i, _B - 1), 0, 0)),
            pl.BlockSpec((1, _L, _K), lambda i: (jnp.maximum(i - _B, 0), 0, 0)),
            pl.BlockSpec((1, _L, _K), lambda i: (0, 0, 0)),
            pl.BlockSpec((1, 1, _L), lambda i: (jnp.maximum(i - _B, 0), 0, 0)),
            pl.BlockSpec((1, _RCHUNK // 128, 128), lambda i: (i, 0, 0)),
            pl.BlockSpec((1, _RCHUNK // 128, 128), lambda i: (i, 0, 0)),
        ],
        out_specs=pl.BlockSpec((1, 1), lambda i: (0, 0)),
        out_shape=jax.ShapeDtypeStruct((1, 1), jnp.float32),
        scratch_shapes=[
            pltpu.VMEM((_L, _K), jnp.float32),
            pltpu.SMEM((1,), jnp.float32),
            pltpu.SMEM((1,), jnp.float32),
            pltpu.SMEM((1,), jnp.float32),
        ],
        compiler_params=pltpu.CompilerParams(
            dimension_semantics=("arbitrary",),
        ),
        interpret=interpret,
    )(alpha, teacher_q, student_q, prototype, mask_f, recon2, label2)
    return out[0, 0]


def kernel(student_Q, teacher_Q, recon, patches_labels, label, epoch,
           prototype):
    alpha = jnp.where(jnp.asarray(epoch, jnp.int32) == 0, 0.0, _MOM)
    alpha = alpha.astype(jnp.float32).reshape(1, 1)
    mask_f = (patches_labels == 0).astype(jnp.float32).reshape(_B, 1, _L)
    recon2 = recon.reshape(_GRID, _RCHUNK // 128, 128)
    label2 = label.reshape(_GRID, _RCHUNK // 128, 128)
    return _afire_loss(alpha, teacher_Q, student_Q, prototype, mask_f,
                       recon2, label2)


# speedup vs baseline: 1.1877x; 1.1877x over previous
import functools

import jax
import jax.numpy as jnp
from jax.experimental import pallas as pl
from jax.experimental.pallas import tpu as pltpu

_B = 32
_L = 196
_K = 4096
_INV_TEMP = 10.0
_MOM = 0.75
_SINKHORN_ITERS = 3
_RECON_N = 32 * 3 * 224 * 224
_GRID = 2 * _B


def _body(alpha_ref, teacher_ref, student_ref, proto_ref, mask_ref,
          recon_ref, label_ref, out_ref, acc_ref, cst_ref, cnt_ref, rec_ref):
    i = pl.program_id(0)

    @pl.when(i == 0)
    def _init():
        cst_ref[0] = 0.0
        cnt_ref[0] = 0.0
        rec_ref[0] = 0.0
        acc_ref[...] = teacher_ref[0]

    @pl.when(jnp.logical_and(i > 0, i < _B))
    def _teacher_acc():
        acc_ref[...] += teacher_ref[0]

    @pl.when(i < _B)
    def _recon():
        dr = recon_ref[0] - label_ref[0]
        rec_ref[0] += jnp.sum(dr * dr)

    @pl.when(i == _B)
    def _sinkhorn():
        alpha = alpha_ref[0, 0]
        batch_q = acc_ref[...] * (1.0 / _B)
        proto = alpha * proto_ref[0] + (1.0 - alpha) * batch_q
        a = jnp.exp(proto * 20.0)
        for _ in range(_SINKHORN_ITERS):
            a = a / jnp.sum(a, axis=0, keepdims=True)
            a = a / jnp.sum(a, axis=1, keepdims=True)
        acc_ref[...] = a

    @pl.when(i >= _B)
    def _student():
        x = student_ref[0] * _INV_TEMP
        t = acc_ref[...]
        m = jnp.max(x, axis=-1)
        se = jnp.sum(jnp.exp(x - m[:, None]), axis=-1)
        lse = m + jnp.log(se)
        dot = jnp.sum(t * x, axis=-1)
        mask = mask_ref[0, 0]
        cst_ref[0] += jnp.sum((lse - dot) * mask)
        cnt_ref[0] += jnp.sum(mask)

    @pl.when(i == _GRID - 1)
    def _finish():
        val = cst_ref[0] / cnt_ref[0] + rec_ref[0] * (1.0 / _RECON_N)
        out_ref[...] = jnp.full((1, 1), val, dtype=jnp.float32)


@functools.partial(jax.jit, static_argnames=("interpret",))
def _afire_loss(alpha, teacher_q, student_q, prototype, mask_f, recon,
                label, interpret=False):
    out = pl.pallas_call(
        _body,
        grid=(_GRID,),
        in_specs=[
            pl.BlockSpec(memory_space=pltpu.SMEM),
            pl.BlockSpec((1, _L, _K), lambda i: (jnp.minimum(i, _B - 1), 0, 0)),
            pl.BlockSpec((1, _L, _K), lambda i: (jnp.maximum(i - _B, 0), 0, 0)),
            pl.BlockSpec((1, _L, _K), lambda i: (0, 0, 0)),
            pl.BlockSpec((1, 1, _L), lambda i: (jnp.maximum(i - _B, 0), 0, 0)),
            pl.BlockSpec((1, 3, 224, 224), lambda i: (jnp.minimum(i, _B - 1), 0, 0, 0)),
            pl.BlockSpec((1, 3, 224, 224), lambda i: (jnp.minimum(i, _B - 1), 0, 0, 0)),
        ],
        out_specs=pl.BlockSpec((1, 1), lambda i: (0, 0)),
        out_shape=jax.ShapeDtypeStruct((1, 1), jnp.float32),
        scratch_shapes=[
            pltpu.VMEM((_L, _K), jnp.float32),
            pltpu.SMEM((1,), jnp.float32),
            pltpu.SMEM((1,), jnp.float32),
            pltpu.SMEM((1,), jnp.float32),
        ],
        compiler_params=pltpu.CompilerParams(
            dimension_semantics=("arbitrary",),
        ),
        interpret=interpret,
    )(alpha, teacher_q, student_q, prototype, mask_f, recon, label)
    return out[0, 0]


def kernel(student_Q, teacher_Q, recon, patches_labels, label, epoch,
           prototype):
    alpha = jnp.where(jnp.asarray(epoch, jnp.int32) == 0, 0.0, _MOM)
    alpha = alpha.astype(jnp.float32).reshape(1, 1)
    mask_f = (patches_labels == 0).astype(jnp.float32).reshape(_B, 1, _L)
    return _afire_loss(alpha, teacher_Q, student_Q, prototype, mask_f,
                       recon, label)


# speedup vs baseline: 2.8581x; 2.4065x over previous
import functools

import jax
import jax.numpy as jnp
from jax.experimental import pallas as pl
from jax.experimental.pallas import tpu as pltpu

_B = 32
_L = 196
_K = 4096
_INV_TEMP = 10.0
_MOM = 0.75
_SINKHORN_ITERS = 3
_RECON_N = 32 * 3 * 224 * 224
_LC = 7
_NCH = _L // _LC
_GRID = 2 * _NCH
_RROWS = _RECON_N // 224
_RBLK = _RROWS // _GRID


def _body(alpha_ref, teacher_ref, student_ref, pt_ref, mask_ref,
          recon_ref, label_ref, out_ref, acc_ref, cst_ref, cnt_ref, rec_ref):
    i = pl.program_id(0)

    @pl.when(i == 0)
    def _init():
        cst_ref[0] = 0.0
        cnt_ref[0] = 0.0
        rec_ref[0] = 0.0

    @pl.when(i < _NCH)
    def _teacher():
        alpha = alpha_ref[0, 0]
        for l in range(_LC):
            s = jnp.sum(teacher_ref[l], axis=0) * (1.0 / _B)
            acc_ref[i, l] = alpha * pt_ref[0, l] + (1.0 - alpha) * s

    dr = recon_ref[...] - label_ref[...]
    rec_ref[0] += jnp.sum(dr * dr)

    @pl.when(i == _NCH)
    def _sinkhorn():
        for c in range(_NCH):
            acc_ref[c] = jnp.exp(acc_ref[c] * 20.0)
        for _ in range(_SINKHORN_ITERS):
            cs = jnp.sum(acc_ref[0], axis=0)
            for c in range(1, _NCH):
                cs = cs + jnp.sum(acc_ref[c], axis=0)
            inv_cs = 1.0 / cs
            for c in range(_NCH):
                a = acc_ref[c] * inv_cs[None, :]
                acc_ref[c] = a / jnp.sum(a, axis=1)[:, None]

    @pl.when(i >= _NCH)
    def _student():
        j = i - _NCH
        msk = mask_ref[0]
        cst = 0.0
        for l in range(_LC):
            x = student_ref[l] * _INV_TEMP
            t = acc_ref[j, l]
            m = jnp.max(x, axis=1)
            se = jnp.sum(jnp.exp(x - m[:, None]), axis=1)
            lse = m + jnp.log(se)
            dot = jnp.sum(t[None, :] * x, axis=1)
            cst = cst + jnp.sum((lse - dot) * msk[l])
        cst_ref[0] += cst
        cnt_ref[0] += jnp.sum(msk)

    @pl.when(i == _GRID - 1)
    def _finish():
        val = cst_ref[0] / cnt_ref[0] + rec_ref[0] * (1.0 / _RECON_N)
        out_ref[...] = jnp.full((1, 1), val, dtype=jnp.float32)


@functools.partial(jax.jit, static_argnames=("interpret",))
def _afire_loss(alpha, teacher_t, student_t, proto2, mask_t, recon2,
                label2, interpret=False):
    out = pl.pallas_call(
        _body,
        grid=(_GRID,),
        in_specs=[
            pl.BlockSpec(memory_space=pltpu.SMEM),
            pl.BlockSpec((_LC, _B, _K), lambda i: (jnp.minimum(i, _NCH - 1), 0, 0)),
            pl.BlockSpec((_LC, _B, _K), lambda i: (jnp.maximum(i - _NCH, 0), 0, 0)),
            pl.BlockSpec((1, _LC, _K), lambda i: (jnp.minimum(i, _NCH - 1), 0, 0)),
            pl.BlockSpec((1, _LC, _B), lambda i: (jnp.maximum(i - _NCH, 0), 0, 0)),
            pl.BlockSpec((_RBLK, 224), lambda i: (i, 0)),
            pl.BlockSpec((_RBLK, 224), lambda i: (i, 0)),
        ],
        out_specs=pl.BlockSpec((1, 1), lambda i: (0, 0)),
        out_shape=jax.ShapeDtypeStruct((1, 1), jnp.float32),
        scratch_shapes=[
            pltpu.VMEM((_NCH, _LC, _K), jnp.float32),
            pltpu.SMEM((1,), jnp.float32),
            pltpu.SMEM((1,), jnp.float32),
            pltpu.SMEM((1,), jnp.float32),
        ],
        compiler_params=pltpu.CompilerParams(
            dimension_semantics=("arbitrary",),
        ),
        interpret=interpret,
    )(alpha, teacher_t, student_t, proto2, mask_t, recon2, label2)
    return out[0, 0]


def kernel(student_Q, teacher_Q, recon, patches_labels, label, epoch,
           prototype):
    alpha = jnp.where(jnp.asarray(epoch, jnp.int32) == 0, 0.0, _MOM)
    alpha = alpha.astype(jnp.float32).reshape(1, 1)
    student_t = student_Q.transpose(1, 0, 2)
    teacher_t = teacher_Q.transpose(1, 0, 2)
    proto2 = prototype.reshape(_NCH, _LC, _K)
    mask_t = (patches_labels == 0).astype(jnp.float32).T.reshape(_NCH, _LC, _B)
    recon2 = recon.reshape(_RROWS, 224)
    label2 = label.reshape(_RROWS, 224)
    return _afire_loss(alpha, teacher_t, student_t, proto2, mask_t,
                       recon2, label2)
